# baseline (speedup 1.0000x reference)
import jax
import jax.numpy as jnp
from jax import lax
from jax.experimental import pallas as pl
from jax.experimental.pallas import tpu as pltpu
from jax.experimental.pallas import tpu_sc as plsc

NC = 2
NS = 16
NW = NC * NS
BATCH = 4096
SEQ = 200
D = 64
SCALE = 8.0
BPW = BATCH // NW
S0, S1 = 96, 104


def _body(seq_hbm, table_hbm, p_hbm, out_hbm,
          idx_v, p_v, in0, in1, out0, out1, g0, g1, w0, w1):
    wid = lax.axis_index("s") * NC + lax.axis_index("c")
    wb0 = wid * BPW

    pltpu.sync_copy(seq_hbm.at[pl.ds(wb0 * SEQ, BPW * SEQ)], idx_v)
    pltpu.sync_copy(p_hbm, p_v)

    ins = [in0, in1]
    outs = [out0, out1]
    gsems = [g0, g1]
    wsems = [w0, w1]

    def start_gather(b, buf, sem):
        pltpu.async_copy(table_hbm.at[idx_v.at[pl.ds(b * SEQ, S0)]],
                         buf.at[pl.ds(0, S0)], sem)
        pltpu.async_copy(table_hbm.at[idx_v.at[pl.ds(b * SEQ + S0, S1)]],
                         buf.at[pl.ds(S0, S1)], sem)

    def wait_gather(b, buf, sem):
        pltpu.make_async_copy(table_hbm.at[idx_v.at[pl.ds(b * SEQ, S0)]],
                              buf.at[pl.ds(0, S0)], sem).wait()
        pltpu.make_async_copy(table_hbm.at[idx_v.at[pl.ds(b * SEQ + S0, S1)]],
                              buf.at[pl.ds(S0, S1)], sem).wait()

    start_gather(0, in0, g0)
    start_gather(1, in1, g1)

    def compute(b, src, dst):
        def row(j, carry):
            iv = plsc.load_gather(idx_v, [jnp.full((16,), b * SEQ + j, jnp.int32)])
            scale = jnp.where(iv == 0, 0.0, SCALE).astype(jnp.float32)
            for k in range(D // 16):
                sl = pl.ds(k * 16, 16)
                dst[j, sl] = src[j, sl] * scale + p_v[j, sl]
            return carry
        lax.fori_loop(0, SEQ, row, 0)

    def step(s, carry):
        for r in range(2):
            b = 2 * s + r
            src, dst, gs, ws = ins[r], outs[r], gsems[r], wsems[r]
            wait_gather(b, src, gs)

            @pl.when(b >= 2)
            def _():
                pltpu.make_async_copy(dst, out_hbm.at[wb0 + b - 2], ws).wait()

            compute(b, src, dst)

            @pl.when(b + 2 < BPW)
            def _():
                start_gather(b + 2, src, gs)

            pltpu.async_copy(dst, out_hbm.at[wb0 + b], ws)
        return carry

    lax.fori_loop(0, BPW // 2, step, 0)

    pltpu.make_async_copy(out0, out_hbm.at[wb0 + BPW - 2], w0).wait()
    pltpu.make_async_copy(out1, out_hbm.at[wb0 + BPW - 1], w1).wait()


def kernel(seq, table, p):
    mesh = plsc.VectorSubcoreMesh(core_axis_name="c", subcore_axis_name="s")
    f = pl.kernel(
        _body,
        out_type=jax.ShapeDtypeStruct((BATCH, SEQ, D), jnp.float32),
        mesh=mesh,
        compiler_params=pltpu.CompilerParams(needs_layout_passes=False,
                                             use_tc_tiling_on_sc=False),
        scratch_types=[
            pltpu.VMEM((BPW * SEQ,), jnp.int32),
            pltpu.VMEM((SEQ, D), jnp.float32),
            pltpu.VMEM((SEQ, D), jnp.float32),
            pltpu.VMEM((SEQ, D), jnp.float32),
            pltpu.VMEM((SEQ, D), jnp.float32),
            pltpu.VMEM((SEQ, D), jnp.float32),
            pltpu.SemaphoreType.DMA,
            pltpu.SemaphoreType.DMA,
            pltpu.SemaphoreType.DMA,
            pltpu.SemaphoreType.DMA,
        ],
    )
    return f(seq.reshape(-1), table, p)
